# baseline (speedup 1.0000x reference)
import functools

import jax
import jax.numpy as jnp
from jax import lax
from jax.experimental import pallas as pl
from jax.experimental.pallas import tpu as pltpu
from jax.experimental.pallas import tpu_sc as plsc

BT = 256
E = 8
D = 2048
I = 2048
TOPK = 2
NP = BT * TOPK
SUB = 64
NPAD = NP + E * SUB
TN = 512
NT = I // TN
TND = 512
ND = D // TND
S = NT + ND

NTILES = 32
GROWS = NPAD // NTILES
TPT = BT // NTILES

_MESH = plsc.VectorSubcoreMesh(core_axis_name="c", subcore_axis_name="s")


def _wid():
    return lax.axis_index("s") * 2 + lax.axis_index("c")


@functools.partial(
    pl.kernel,
    out_type=(
        jax.ShapeDtypeStruct((NPAD,), jnp.int32),
        jax.ShapeDtypeStruct((NPAD,), jnp.float32),
        jax.ShapeDtypeStruct((BT,), jnp.int32),
        jax.ShapeDtypeStruct((BT,), jnp.int32),
        jax.ShapeDtypeStruct((16,), jnp.int32),
        jax.ShapeDtypeStruct((16,), jnp.int32),
        jax.ShapeDtypeStruct((16,), jnp.int32),
    ),
    mesh=_MESH,
    compiler_params=pltpu.CompilerParams(needs_layout_passes=False),
    scratch_types=[
        pltpu.VMEM((NP,), jnp.int32),
        pltpu.VMEM((NP,), jnp.float32),
        pltpu.VMEM((NP,), jnp.int32),
        pltpu.VMEM((NPAD,), jnp.int32),
        pltpu.VMEM((NPAD,), jnp.float32),
        pltpu.VMEM((BT,), jnp.int32),
        pltpu.VMEM((BT,), jnp.int32),
        pltpu.VMEM((16,), jnp.int32),
        pltpu.VMEM((16,), jnp.int32),
        pltpu.VMEM((16,), jnp.int32),
        pltpu.VMEM((16,), jnp.int32),
        pltpu.VMEM((16,), jnp.int32),
    ],
)
def _route(keys_hbm, rwf_hbm, thalf_hbm, perm_hbm, wsort_hbm, pca_hbm, pcb_hbm,
           ro_hbm, ns_hbm, fe_hbm,
           keys_v, rw_v, thalf_v, perm_v, ws_v, pca_v, pcb_v, cnt_v, ro_v, ns_v, fe_v, tmp_v):
    @pl.when(_wid() == 0)
    def _():
        pltpu.sync_copy(keys_hbm, keys_v)
        pltpu.sync_copy(rwf_hbm, rw_v)
        pltpu.sync_copy(thalf_hbm, thalf_v)
        iota = lax.iota(jnp.int32, 16)
        ones = jnp.ones((16,), jnp.int32)
        zi = jnp.zeros((16,), jnp.int32)
        zf = jnp.zeros((16,), jnp.float32)

        def zero_body(j, _):
            perm_v[pl.ds(j * 16, 16)] = zi
            ws_v[pl.ds(j * 16, 16)] = zf
            return 0

        lax.fori_loop(0, NPAD // 16, zero_body, 0)
        cnt_v[...] = zi

        fifteen = jnp.full((16,), 15, jnp.int32)

        def _splat_last(v):
            tmp_v[...] = v
            return plsc.load_gather(tmp_v, [fifteen])

        def _hist_update(k):
            upd = zi
            for e in range(E):
                cs = plsc.cumsum((k == e).astype(jnp.int32))
                upd = jnp.where(iota == e, _splat_last(cs), upd)
            return upd

        def count_body(j, _):
            k = keys_v[pl.ds(j * 16, 16)]
            cnt_v[...] = cnt_v[...] + _hist_update(k)
            return 0

        lax.fori_loop(0, NP // 16, count_body, 0)

        c = cnt_v[...]
        rc = lax.shift_left(
            lax.shift_right_logical(c + jnp.full((16,), SUB - 1, jnp.int32),
                                    jnp.full((16,), 6, jnp.int32)),
            jnp.full((16,), 6, jnp.int32))
        rcs = plsc.cumsum(rc)
        ro_v[...] = rcs - rc
        ns_v[...] = lax.shift_right_logical(rc, jnp.full((16,), 6, jnp.int32))
        fe_v[...] = plsc.cummax(jnp.where(c > 0, iota, zi))
        cnt_v[...] = zi

        def place_body(j, _):
            k = keys_v[pl.ds(j * 16, 16)]
            w = rw_v[pl.ds(j * 16, 16)]
            t = thalf_v[pl.ds(j * 16, 16)]
            bases = plsc.load_gather(cnt_v, [k])
            ro_g = plsc.load_gather(ro_v, [k])
            exc = zi
            for e in range(E):
                m = k == e
                cs = plsc.cumsum(m.astype(jnp.int32))
                exc = jnp.where(m, cs - ones, exc)
            dest = ro_g + bases + exc
            par = jnp.bitwise_and(iota, ones)
            plsc.store_scatter(perm_v, [dest], t)
            plsc.store_scatter(ws_v, [dest], w)
            plsc.store_scatter(pca_v, [t], dest, mask=par == 0)
            plsc.store_scatter(pcb_v, [t], dest, mask=par == 1)
            cnt_v[...] = cnt_v[...] + _hist_update(k)
            return 0

        lax.fori_loop(0, NP // 16, place_body, 0)

        pltpu.sync_copy(perm_v, perm_hbm)
        pltpu.sync_copy(ws_v, wsort_hbm)
        pltpu.sync_copy(pca_v, pca_hbm)
        pltpu.sync_copy(pcb_v, pcb_hbm)
        pltpu.sync_copy(ro_v, ro_hbm)
        pltpu.sync_copy(ns_v, ns_hbm)
        pltpu.sync_copy(fe_v, fe_hbm)


@functools.partial(
    pl.kernel,
    out_type=jax.ShapeDtypeStruct((NPAD, D), jnp.float32),
    mesh=_MESH,
    compiler_params=pltpu.CompilerParams(needs_layout_passes=False),
    scratch_types=[
        pltpu.VMEM((GROWS,), jnp.int32),
        pltpu.VMEM((GROWS, D), jnp.float32),
        pltpu.SemaphoreType.DMA,
    ],
)
def _gather(x_hbm, perm_hbm, xc_hbm, idx_v, rows_v, sem):
    base = _wid() * GROWS
    pltpu.sync_copy(perm_hbm.at[pl.ds(base, GROWS)], idx_v)
    pltpu.async_copy(x_hbm.at[idx_v], rows_v, sem).wait()
    pltpu.sync_copy(rows_v, xc_hbm.at[pl.ds(base, GROWS)])


@functools.partial(
    pl.kernel,
    out_type=jax.ShapeDtypeStruct((BT, D), jnp.float32),
    mesh=_MESH,
    compiler_params=pltpu.CompilerParams(needs_layout_passes=False),
    scratch_types=[
        pltpu.VMEM((TPT,), jnp.int32),
        pltpu.VMEM((TPT,), jnp.int32),
        pltpu.VMEM((TPT, D), jnp.float32),
        pltpu.VMEM((TPT, D), jnp.float32),
        pltpu.SemaphoreType.DMA,
        pltpu.SemaphoreType.DMA,
    ],
)
def _combine(ysc_hbm, pca_hbm, pcb_hbm, out_hbm, ia_v, ib_v, ra_v, rb_v, sa, sb):
    base = _wid() * TPT
    pltpu.sync_copy(pca_hbm.at[pl.ds(base, TPT)], ia_v)
    pltpu.sync_copy(pcb_hbm.at[pl.ds(base, TPT)], ib_v)
    ca = pltpu.async_copy(ysc_hbm.at[ia_v], ra_v, sa)
    cb = pltpu.async_copy(ysc_hbm.at[ib_v], rb_v, sb)
    ca.wait()
    cb.wait()

    def add_body(j, _):
        sl = pl.ds(j * 16, 16)
        for r in range(TPT):
            ra_v[r, sl] = ra_v[r, sl] + rb_v[r, sl]
        return 0

    lax.fori_loop(0, D // 16, add_body, 0)
    pltpu.sync_copy(ra_v, out_hbm.at[pl.ds(base, TPT)])


def _moe_body(fe_ref, rons_ref, xc_ref, wu_ref, wg_ref, w2_ref, ws_ref,
              y_ref, h_ref):
    e = pl.program_id(0)
    s = pl.program_id(1)
    ro = rons_ref[0, e]
    ns = rons_ref[1, e]

    @pl.when(s < NT)
    def _():
        n = s
        wu = wu_ref[0]
        wg = wg_ref[0]

        def body(i, _):
            r0 = pl.multiple_of(ro + i * SUB, SUB)
            x = xc_ref[pl.ds(r0, SUB), :]
            u = jax.lax.dot_general(x, wu, (((1,), (1,)), ((), ())),
                                    preferred_element_type=jnp.float32)
            g = jax.lax.dot_general(x, wg, (((1,), (1,)), ((), ())),
                                    preferred_element_type=jnp.float32)
            h_ref[pl.ds(pl.multiple_of(i * SUB, SUB), SUB), pl.ds(n * TN, TN)] = (u * jax.nn.sigmoid(u)) * g
            return 0

        jax.lax.fori_loop(0, ns, body, 0)

    @pl.when(s >= NT)
    def _():
        nd = s - NT
        w2t = w2_ref[0]

        def body(i, _):
            r0 = pl.multiple_of(ro + i * SUB, SUB)
            h = h_ref[pl.ds(pl.multiple_of(i * SUB, SUB), SUB), :]
            y = jax.lax.dot_general(h, w2t, (((1,), (1,)), ((), ())),
                                    preferred_element_type=jnp.float32)
            w = ws_ref[pl.ds(r0, SUB), :]
            y_ref[pl.ds(r0, SUB), pl.ds(nd * TND, TND)] = y * w
            return 0

        jax.lax.fori_loop(0, ns, body, 0)


def kernel(hidden_states, expert_routing_table, router_weights, w13_weight, w2_weight):
    x = hidden_states.astype(jnp.float32)
    keys = expert_routing_table.astype(jnp.int32).reshape(-1)
    rwf = router_weights.astype(jnp.float32).reshape(-1)

    thalf = (jnp.arange(NP, dtype=jnp.int32) // TOPK)
    perm, wsort, pca, pcb, ro16, ns16, fe16 = _route(keys, rwf, thalf)
    xc = _gather(x, perm)

    rons = jnp.stack([ro16[:E], ns16[:E]])
    fe = fe16[:E]
    ws2 = wsort[:, None]

    grid_spec = pltpu.PrefetchScalarGridSpec(
        num_scalar_prefetch=2,
        grid=(E, S),
        in_specs=[
            pl.BlockSpec((NPAD, D), lambda e, s, fe, rons: (0, 0)),
            pl.BlockSpec((1, TN, D), lambda e, s, fe, rons: (fe[e], jnp.minimum(s, NT - 1), 0)),
            pl.BlockSpec((1, TN, D), lambda e, s, fe, rons: (fe[e], NT + jnp.minimum(s, NT - 1), 0)),
            pl.BlockSpec((1, TND, D), lambda e, s, fe, rons: (fe[e], jnp.maximum(s - NT, 0), 0)),
            pl.BlockSpec((NPAD, 1), lambda e, s, fe, rons: (0, 0)),
        ],
        out_specs=pl.BlockSpec((NPAD, D), lambda e, s, fe, rons: (0, 0)),
        scratch_shapes=[pltpu.VMEM((BT, I), jnp.float32)],
    )
    ysc = pl.pallas_call(
        _moe_body,
        grid_spec=grid_spec,
        out_shape=jax.ShapeDtypeStruct((NPAD, D), jnp.float32),
        compiler_params=pltpu.CompilerParams(vmem_limit_bytes=56 * 1024 * 1024),
    )(fe, rons, xc, w13_weight, w13_weight, w2_weight, ws2)

    return _combine(ysc, pca, pcb)


# speedup vs baseline: 1.4705x; 1.4705x over previous
import functools

import jax
import jax.numpy as jnp
from jax import lax
from jax.experimental import pallas as pl
from jax.experimental.pallas import tpu as pltpu
from jax.experimental.pallas import tpu_sc as plsc

BT = 256
E = 8
D = 2048
I = 2048
TOPK = 2
NP = BT * TOPK
TN = 512
NT = I // TN
TND = 512
ND = D // TND
S = NT + ND

NTILES = 32
TPT = BT // NTILES

_MESH = plsc.VectorSubcoreMesh(core_axis_name="c", subcore_axis_name="s")


def _wid():
    return lax.axis_index("s") * 2 + lax.axis_index("c")


def _moe_body(x_ref, ert_ref, rw_ref, wu_ref, wg_ref, w2_ref, y_ref, h_ref):
    e = pl.program_id(0)
    s = pl.program_id(1)

    @pl.when(s < NT)
    def _():
        x = x_ref[...]
        u = jax.lax.dot_general(x, wu_ref[0], (((1,), (1,)), ((), ())),
                                preferred_element_type=jnp.float32)
        g = jax.lax.dot_general(x, wg_ref[0], (((1,), (1,)), ((), ())),
                                preferred_element_type=jnp.float32)
        h_ref[:, pl.ds(s * TN, TN)] = (u * jax.nn.sigmoid(u)) * g

    @pl.when(s >= NT)
    def _():
        y = jax.lax.dot_general(h_ref[...], w2_ref[0], (((1,), (1,)), ((), ())),
                                preferred_element_type=jnp.float32)
        sel = (ert_ref[...] == e).astype(jnp.float32) * rw_ref[...]
        we = jnp.sum(sel, axis=1, keepdims=True)
        y_ref[0] = y * we


@functools.partial(
    pl.kernel,
    out_type=jax.ShapeDtypeStruct((BT, D), jnp.float32),
    mesh=_MESH,
    compiler_params=pltpu.CompilerParams(needs_layout_passes=False),
    scratch_types=[
        pltpu.VMEM((16,), jnp.int32),
        pltpu.VMEM((TPT,), jnp.int32),
        pltpu.VMEM((TPT,), jnp.int32),
        pltpu.VMEM((TPT, D), jnp.float32),
        pltpu.VMEM((TPT, D), jnp.float32),
        pltpu.SemaphoreType.DMA,
        pltpu.SemaphoreType.DMA,
    ],
)
def _combine(ysc_hbm, keys_hbm, out_hbm, k_v, ia_v, ib_v, ra_v, rb_v, sa, sb):
    base = _wid() * TPT
    pltpu.sync_copy(keys_hbm.at[pl.ds(base * TOPK, 16)], k_v)
    iota = lax.iota(jnp.int32, 16)
    ones = jnp.ones((16,), jnp.int32)
    half = lax.shift_right_logical(iota, ones)
    tok = jnp.full((16,), base, jnp.int32) + half
    pc = k_v[...] * jnp.full((16,), BT, jnp.int32) + tok
    even = jnp.bitwise_and(iota, ones) == 0
    plsc.store_scatter(ia_v, [half], pc, mask=even)
    plsc.store_scatter(ib_v, [half], pc, mask=jnp.logical_not(even))
    ca = pltpu.async_copy(ysc_hbm.at[ia_v], ra_v, sa)
    cb = pltpu.async_copy(ysc_hbm.at[ib_v], rb_v, sb)
    ca.wait()
    cb.wait()

    def add_body(j, _):
        sl = pl.ds(j * 16, 16)
        for r in range(TPT):
            ra_v[r, sl] = ra_v[r, sl] + rb_v[r, sl]
        return 0

    lax.fori_loop(0, D // 16, add_body, 0)
    pltpu.sync_copy(ra_v, out_hbm.at[pl.ds(base, TPT)])


def kernel(hidden_states, expert_routing_table, router_weights, w13_weight, w2_weight):
    x = hidden_states.astype(jnp.float32)
    ert = expert_routing_table.astype(jnp.int32)
    rw = router_weights.astype(jnp.float32)

    ysc = pl.pallas_call(
        _moe_body,
        grid=(E, S),
        in_specs=[
            pl.BlockSpec((BT, D), lambda e, s: (0, 0)),
            pl.BlockSpec((BT, 2), lambda e, s: (0, 0)),
            pl.BlockSpec((BT, 2), lambda e, s: (0, 0)),
            pl.BlockSpec((1, TN, D), lambda e, s: (e, jnp.minimum(s, NT - 1), 0)),
            pl.BlockSpec((1, TN, D), lambda e, s: (e, NT + jnp.minimum(s, NT - 1), 0)),
            pl.BlockSpec((1, TND, D), lambda e, s: (e, jnp.maximum(s - NT, 0), 0)),
        ],
        out_specs=pl.BlockSpec((1, BT, TND), lambda e, s: (e, 0, jnp.maximum(s - NT, 0))),
        out_shape=jax.ShapeDtypeStruct((E, BT, D), jnp.float32),
        scratch_shapes=[pltpu.VMEM((BT, I), jnp.float32)],
        compiler_params=pltpu.CompilerParams(vmem_limit_bytes=56 * 1024 * 1024),
    )(x, ert, rw, w13_weight, w13_weight, w2_weight)

    keys = ert.reshape(-1)
    return _combine(ysc.reshape(E * BT, D), keys)


# speedup vs baseline: 1.4863x; 1.0107x over previous
import functools

import jax
import jax.numpy as jnp
from jax import lax
from jax.experimental import pallas as pl
from jax.experimental.pallas import tpu as pltpu
from jax.experimental.pallas import tpu_sc as plsc

BT = 256
E = 8
D = 2048
I = 2048
TOPK = 2
NP = BT * TOPK
TN = 512
NT = I // TN
TND = 512
ND = D // TND
S = NT + ND

NTILES = 32
TPT = BT // NTILES

_MESH = plsc.VectorSubcoreMesh(core_axis_name="c", subcore_axis_name="s")


def _wid():
    return lax.axis_index("s") * 2 + lax.axis_index("c")


def _moe_body(x_ref, ert_ref, rw_ref, wu_ref, wg_ref, w2_ref, y_ref, h_ref):
    e = pl.program_id(0)
    s = pl.program_id(1)

    @pl.when(s < NT)
    def _():
        x = x_ref[...]
        u = jax.lax.dot_general(x, wu_ref[0], (((1,), (1,)), ((), ())),
                                preferred_element_type=jnp.float32)
        g = jax.lax.dot_general(x, wg_ref[0], (((1,), (1,)), ((), ())),
                                preferred_element_type=jnp.float32)
        h_ref[:, pl.ds(s * TN, TN)] = (u * jax.nn.sigmoid(u)) * g

    @pl.when(s >= NT)
    def _():
        y = jax.lax.dot_general(h_ref[...], w2_ref[0], (((1,), (1,)), ((), ())),
                                preferred_element_type=jnp.float32)
        sel = (ert_ref[...] == e).astype(jnp.float32) * rw_ref[...]
        we = jnp.sum(sel, axis=1, keepdims=True)
        y_ref[0] = y * we


@functools.partial(
    pl.kernel,
    out_type=jax.ShapeDtypeStruct((BT, D), jnp.float32),
    mesh=_MESH,
    compiler_params=pltpu.CompilerParams(needs_layout_passes=False),
    scratch_types=[
        pltpu.VMEM((16,), jnp.int32),
        pltpu.VMEM((TPT,), jnp.int32),
        pltpu.VMEM((TPT,), jnp.int32),
        pltpu.VMEM((TPT, D), jnp.float32),
        pltpu.VMEM((TPT, D), jnp.float32),
        pltpu.SemaphoreType.DMA,
        pltpu.SemaphoreType.DMA,
    ],
)
def _combine(ysc_hbm, keys_hbm, out_hbm, k_v, ia_v, ib_v, ra_v, rb_v, sa, sb):
    base = _wid() * TPT
    pltpu.sync_copy(keys_hbm.at[pl.ds(base * TOPK, 16)], k_v)
    iota = lax.iota(jnp.int32, 16)
    ones = jnp.ones((16,), jnp.int32)
    half = lax.shift_right_logical(iota, ones)
    tok = jnp.full((16,), base, jnp.int32) + half
    pc = k_v[...] * jnp.full((16,), BT, jnp.int32) + tok
    even = jnp.bitwise_and(iota, ones) == 0
    plsc.store_scatter(ia_v, [half], pc, mask=even)
    plsc.store_scatter(ib_v, [half], pc, mask=jnp.logical_not(even))
    ca = pltpu.async_copy(ysc_hbm.at[ia_v], ra_v, sa)
    cb = pltpu.async_copy(ysc_hbm.at[ib_v], rb_v, sb)
    ca.wait()
    cb.wait()

    def add_body(j, _):
        sl = pl.ds(j * 16, 16)
        for r in range(TPT):
            ra_v[r, sl] = ra_v[r, sl] + rb_v[r, sl]
        return 0

    lax.fori_loop(0, D // 16, add_body, 0)
    pltpu.sync_copy(ra_v, out_hbm.at[pl.ds(base, TPT)])


def kernel(hidden_states, expert_routing_table, router_weights, w13_weight, w2_weight):
    x = hidden_states.astype(jnp.float32)
    ert = expert_routing_table.astype(jnp.int32)
    rw = router_weights.astype(jnp.float32)

    ysc = pl.pallas_call(
        _moe_body,
        grid=(E, S),
        in_specs=[
            pl.BlockSpec((BT, D), lambda e, s: (0, 0)),
            pl.BlockSpec((BT, 2), lambda e, s: (0, 0)),
            pl.BlockSpec((BT, 2), lambda e, s: (0, 0)),
            pl.BlockSpec((1, TN, D),
                         lambda e, s: (jnp.where(s < NT, e, jnp.minimum(e + 1, E - 1)),
                                       jnp.where(s < NT, s, 0), 0)),
            pl.BlockSpec((1, TN, D),
                         lambda e, s: (jnp.where(s < NT, e, jnp.minimum(e + 1, E - 1)),
                                       NT + jnp.where(s < NT, s, 0), 0)),
            pl.BlockSpec((1, TND, D),
                         lambda e, s: (jnp.where(s >= NT - 1, e, jnp.maximum(e - 1, 0)),
                                       jnp.where(s >= NT - 1,
                                                 jnp.clip(s - NT, 0, ND - 1),
                                                 jnp.where(e > 0, ND - 1, 0)), 0)),
        ],
        out_specs=pl.BlockSpec((1, BT, TND), lambda e, s: (e, 0, jnp.maximum(s - NT, 0))),
        out_shape=jax.ShapeDtypeStruct((E, BT, D), jnp.float32),
        scratch_shapes=[pltpu.VMEM((BT, I), jnp.float32)],
        compiler_params=pltpu.CompilerParams(vmem_limit_bytes=56 * 1024 * 1024),
    )(x, ert, rw, w13_weight, w13_weight, w2_weight)

    keys = ert.reshape(-1)
    return _combine(ysc.reshape(E * BT, D), keys)
